# baseline (speedup 1.0000x reference)
import functools

import jax
import jax.numpy as jnp
from jax import lax
from jax.experimental import pallas as pl
from jax.experimental.pallas import tpu as pltpu
from jax.experimental.pallas import tpu_sc as plsc

_STRENGTH_DECAY = 0.99
_AGE_INCREMENT = 0.02
_TEMPERATURE = 0.1
_EPS = 1e-6

_BLK = 16
_NWORKERS = 32
_NCORES = 2


def _tc_body(sig_ref, hid_ref, keys_ref, str_ref, age_ref,
             wk_ref, bk_ref, wv_ref, bv_ref, wwg_ref, bwg_ref,
             wpg_ref, bpg_ref, wmg_ref, bmg_ref,
             okeys_ref, ostr_ref, oage_ref,
             fidx_ref, am_ref, cva_ref):
    f32 = jnp.float32
    sig = sig_ref[...]
    hid = hid_ref[...]
    joined = jnp.concatenate([sig, hid], axis=-1)

    ck = jnp.tanh(
        lax.dot_general(joined, wk_ref[...], (((1,), (1,)), ((), ())),
                        preferred_element_type=f32) + bk_ref[...])
    cv = jnp.tanh(
        lax.dot_general(joined, wv_ref[...], (((1,), (1,)), ((), ())),
                        preferred_element_type=f32) + bv_ref[...])
    ws = jax.nn.sigmoid(
        jnp.sum(joined * wwg_ref[...], axis=-1, keepdims=True) + bwg_ref[...])
    pg = jax.nn.sigmoid(
        jnp.sum(joined * wpg_ref[...], axis=-1, keepdims=True) + bpg_ref[...])
    mg_lin = jnp.sum(joined * wmg_ref[...], axis=-1, keepdims=True) + bmg_ref[...]

    cnorm = jnp.sqrt(jnp.sum(ck * ck, axis=-1, keepdims=True))
    ncand = ck / jnp.maximum(cnorm, _EPS)

    keys = keys_ref[...]
    R, N, _ = keys.shape
    keysq = jnp.sum(keys * keys, axis=-1)
    keynorm = jnp.maximum(jnp.sqrt(keysq), _EPS)
    dots = jnp.sum(keys * ncand[:, None, :], axis=-1)
    sim = dots / keynorm

    iota = lax.broadcasted_iota(jnp.int32, sim.shape, 1)
    neg = jnp.float32(-jnp.inf)

    s1 = jnp.max(sim, axis=-1, keepdims=True)
    i1 = jnp.argmax(sim, axis=-1, keepdims=True)
    sim_m = jnp.where(iota == i1, neg, sim)
    s2 = jnp.max(sim_m, axis=-1, keepdims=True)
    i2 = jnp.argmax(sim_m, axis=-1, keepdims=True)
    sim_m = jnp.where(iota == i2, neg, sim_m)
    s3 = jnp.max(sim_m, axis=-1, keepdims=True)
    i3 = jnp.argmax(sim_m, axis=-1, keepdims=True)

    strength = str_ref[...]
    age = age_ref[...]
    replace_scores = 1.2 * age + 1.0 * (1.0 - strength) + 0.5 * (1.0 - sim)
    ri = jnp.argmax(replace_scores, axis=-1, keepdims=True)

    novelty = jnp.clip(1.0 - s1, 0.0, 1.0)
    merge_pref = jax.nn.sigmoid(mg_lin + 2.6 * s1)
    full_m = (s1 > 0.78) & (merge_pref >= 0.55)
    multi_m = full_m & (s2 > 0.68)
    partial_m = (~multi_m) & (s1 > 0.64) & (s2 > 0.52)

    e2 = jnp.exp((s2 - s1) / _TEMPERATURE)
    e3 = jnp.exp((s3 - s1) / _TEMPERATURE)
    pden = 1.0 + e2
    pw1 = 1.0 / pden
    pw2 = e2 / pden
    mden = 1.0 + e2 + e3
    mw1 = 1.0 / mden
    mw2 = e2 / mden
    mw3 = e3 / mden

    oh1 = (iota == i1).astype(f32)
    oh2 = (iota == i2).astype(f32)
    oh3 = (iota == i3).astype(f32)
    tw = (iota == ri).astype(f32)
    tw = jnp.where(full_m, oh1, tw)
    tw = jnp.where(partial_m, pw1 * oh1 + pw2 * oh2, tw)
    tw = jnp.where(multi_m, mw1 * oh1 + mw2 * oh2 + mw3 * oh3, tw)

    scale = jnp.where(multi_m, 0.16 + 0.52 * ws,
                      jnp.where(partial_m, 0.18 + 0.62 * ws,
                                0.2 + 0.8 * ws))
    fac = scale * (0.55 + 0.45 * novelty)
    ow = tw * fac

    merge_like = full_m | partial_m | multi_m
    kmix = jnp.where(merge_like, 0.28 + 0.24 * pg, 0.78 + 0.16 * pg)
    vmix = jnp.where(merge_like, 0.42 + 0.28 * pg, 0.82 + 0.12 * pg)

    owk = (ow * kmix)[:, :, None]
    okeys_ref[...] = keys + owk * (ck[:, None, :] - keys)

    boost = ow * (0.45 + 0.35 * pg + 0.45 * novelty + 0.25 * ws)
    ostr_ref[...] = jnp.clip(strength * _STRENGTH_DECAY + boost, 0.0, 1.0)
    oage_ref[...] = jnp.clip((age + _AGE_INCREMENT) * (1.0 - ow), 0.0, 1.0)

    zero = jnp.zeros_like(s1)
    t1 = jnp.where(multi_m, mw1,
                   jnp.where(partial_m, pw1,
                             jnp.where(full_m, jnp.ones_like(s1), zero)))
    t2 = jnp.where(multi_m, mw2, jnp.where(partial_m, pw2, zero))
    t3 = jnp.where(multi_m, mw3, zero)
    tr = jnp.where(merge_like, zero, jnp.ones_like(s1))
    fv = fac * vmix
    c1, c2, c3, cr = t1 * fv, t2 * fv, t3 * fv, tr * fv

    def _used(c):
        return (i1 == c) | (i2 == c) | (i3 == c) | (ri == c)
    s_sent = jnp.where(~_used(0), 0,
                       jnp.where(~_used(1), 1,
                                 jnp.where(~_used(2), 2,
                                           jnp.where(~_used(3), 3, 4))))
    s_sent = s_sent.astype(jnp.int32)

    row_id = (pl.program_id(0) * R
              + lax.broadcasted_iota(jnp.int32, (R, 1), 0))
    base = row_id * N
    j1 = base + jnp.where(c1 > 0, i1, s_sent)
    j2 = base + jnp.where(c2 > 0, i2, s_sent)
    j3 = base + jnp.where(c3 > 0, i3, s_sent)
    j4 = base + jnp.where(cr > 0, ri, s_sent)
    jp = base + s_sent
    fidx_ref[...] = jnp.concatenate(
        [j1, j2, j3, j4, jp, jp, jp, jp], axis=1)

    z = jnp.zeros_like(c1)
    cs = jnp.concatenate([c1, c2, c3, cr, z, z, z, z], axis=1)
    D_ = cv.shape[-1]
    am_ref[...] = jnp.broadcast_to((1.0 - cs)[:, :, None], (R, 8, D_))
    cva_ref[...] = cs[:, :, None] * cv[:, None, :]


def _make_sc_copy(n_rows, D, interpret=False):
    rows_per_w = n_rows // _NWORKERS
    mesh = plsc.VectorSubcoreMesh(core_axis_name="c", subcore_axis_name="s",
                                  num_cores=_NCORES,
                                  num_subcores=_NWORKERS // _NCORES)

    @functools.partial(
        pl.kernel,
        out_type=jax.ShapeDtypeStruct((n_rows, D), jnp.float32),
        mesh=mesh,
        interpret=interpret,
        scratch_types=[pltpu.SemaphoreType.DMA],
    )
    def sc_copy(vflat_hbm, out_ref, sem):
        wid = lax.axis_index("s") * _NCORES + lax.axis_index("c")
        base = wid * rows_per_w
        sl = pl.ds(base, rows_per_w)
        pltpu.async_copy(vflat_hbm.at[sl], out_ref.at[sl], sem).wait()

    return sc_copy


def _make_sc_fixup(n_entries, D, interpret=False):
    per_w = n_entries // _NWORKERS
    mesh = plsc.VectorSubcoreMesh(core_axis_name="c", subcore_axis_name="s",
                                  num_cores=_NCORES,
                                  num_subcores=_NWORKERS // _NCORES)

    @functools.partial(
        pl.kernel,
        out_type=(),
        mesh=mesh,
        interpret=interpret,
        scratch_types=[
            pltpu.VMEM((per_w,), jnp.int32),
            pltpu.VMEM((per_w, D), jnp.float32),
            pltpu.VMEM((per_w, D), jnp.float32),
            pltpu.VMEM((per_w, D), jnp.float32),
            pltpu.SemaphoreType.DMA,
        ],
    )
    def sc_fixup(vflat_hbm, fidx_hbm, am_hbm, cva_hbm, out_ref,
                 idx_v, rows_v, am_v, cva_v, sem):
        wid = lax.axis_index("s") * _NCORES + lax.axis_index("c")
        base = wid * per_w
        pltpu.sync_copy(fidx_hbm.at[pl.ds(base, per_w)], idx_v)
        pltpu.async_copy(vflat_hbm.at[idx_v], rows_v, sem).wait()
        pltpu.sync_copy(am_hbm.at[pl.ds(base, per_w)], am_v)
        pltpu.sync_copy(cva_hbm.at[pl.ds(base, per_w)], cva_v)

        def body(e, carry):
            for j in range(D // 16):
                sl = (e, pl.ds(j * 16, 16))
                rows_v[sl] = am_v[sl] * rows_v[sl] + cva_v[sl]
            return carry

        lax.fori_loop(0, per_w, body, 0)
        pltpu.async_copy(rows_v, out_ref.at[idx_v], sem).wait()

    return sc_fixup


def kernel(signal, hidden, episodic_keys, episodic_values, episodic_strength,
           episodic_age, Wk, bk, Wv, bv, Wwg, bwg, Wpg, bpg, Wmg, bmg,
           interpret=False):
    B, N, D = episodic_keys.shape
    R = _BLK
    grid = (B // R,)

    row = lambda i: (i, 0)
    row3 = lambda i: (i, 0, 0)
    const2 = lambda i: (0, 0)

    in_specs = [
        pl.BlockSpec((R, D), row),
        pl.BlockSpec((R, D), row),
        pl.BlockSpec((R, N, D), row3),
        pl.BlockSpec((R, N), row),
        pl.BlockSpec((R, N), row),
        pl.BlockSpec((D, 2 * D), const2),
        pl.BlockSpec((1, D), const2),
        pl.BlockSpec((D, 2 * D), const2),
        pl.BlockSpec((1, D), const2),
        pl.BlockSpec((1, 2 * D), const2),
        pl.BlockSpec((1, 1), const2),
        pl.BlockSpec((1, 2 * D), const2),
        pl.BlockSpec((1, 1), const2),
        pl.BlockSpec((1, 2 * D), const2),
        pl.BlockSpec((1, 1), const2),
    ]
    out_specs = [
        pl.BlockSpec((R, N, D), row3),
        pl.BlockSpec((R, N), row),
        pl.BlockSpec((R, N), row),
        pl.BlockSpec((R, 8), row),
        pl.BlockSpec((R, 8, D), lambda i: (i, 0, 0)),
        pl.BlockSpec((R, 8, D), lambda i: (i, 0, 0)),
    ]
    out_shapes = [
        jax.ShapeDtypeStruct((B, N, D), jnp.float32),
        jax.ShapeDtypeStruct((B, N), jnp.float32),
        jax.ShapeDtypeStruct((B, N), jnp.float32),
        jax.ShapeDtypeStruct((B, 8), jnp.int32),
        jax.ShapeDtypeStruct((B, 8, D), jnp.float32),
        jax.ShapeDtypeStruct((B, 8, D), jnp.float32),
    ]

    vflat = episodic_values.reshape(B * N, D)
    sc_copy = _make_sc_copy(B * N, D, interpret=interpret)
    vcopy = sc_copy(vflat)

    okeys, ostr, oage, fidx, am3, cva3 = pl.pallas_call(
        _tc_body,
        grid=grid,
        in_specs=in_specs,
        out_specs=out_specs,
        out_shape=out_shapes,
        compiler_params=pltpu.CompilerParams(
            dimension_semantics=("parallel",)),
        interpret=interpret,
    )(signal, hidden, episodic_keys,
      episodic_strength, episodic_age,
      Wk, bk.reshape(1, D), Wv, bv.reshape(1, D),
      Wwg, bwg.reshape(1, 1), Wpg, bpg.reshape(1, 1), Wmg, bmg.reshape(1, 1))

    fidx_flat = fidx.reshape(8 * B)
    am_flat = am3.reshape(8 * B, D)
    cva_flat = cva3.reshape(8 * B, D)

    sc_fixup = _make_sc_fixup(8 * B, D, interpret=interpret)
    vref = jax.new_ref(vcopy)
    sc_fixup(vflat, fidx_flat, am_flat, cva_flat, vref)
    updated_values = vref[...].reshape(B, N, D)

    return okeys, updated_values, ostr, oage


# speedup vs baseline: 18.3405x; 18.3405x over previous
import functools

import jax
import jax.numpy as jnp
from jax import lax
from jax.experimental import pallas as pl
from jax.experimental.pallas import tpu as pltpu
from jax.experimental.pallas import tpu_sc as plsc

_STRENGTH_DECAY = 0.99
_AGE_INCREMENT = 0.02
_TEMPERATURE = 0.1
_EPS = 1e-6

_BLK = 16
_NWORKERS = 32
_NCORES = 2


def _tc_body(sig_ref, hid_ref, keys_ref, vals_ref, str_ref, age_ref,
             wk_ref, bk_ref, wv_ref, bv_ref, wwg_ref, bwg_ref,
             wpg_ref, bpg_ref, wmg_ref, bmg_ref,
             okeys_ref, ovals_ref, ostr_ref, oage_ref,
             fidx_ref, am_ref, cva_ref):
    f32 = jnp.float32
    sig = sig_ref[...]
    hid = hid_ref[...]
    joined = jnp.concatenate([sig, hid], axis=-1)

    ck = jnp.tanh(
        lax.dot_general(joined, wk_ref[...], (((1,), (1,)), ((), ())),
                        preferred_element_type=f32) + bk_ref[...])
    cv = jnp.tanh(
        lax.dot_general(joined, wv_ref[...], (((1,), (1,)), ((), ())),
                        preferred_element_type=f32) + bv_ref[...])
    ws = jax.nn.sigmoid(
        jnp.sum(joined * wwg_ref[...], axis=-1, keepdims=True) + bwg_ref[...])
    pg = jax.nn.sigmoid(
        jnp.sum(joined * wpg_ref[...], axis=-1, keepdims=True) + bpg_ref[...])
    mg_lin = jnp.sum(joined * wmg_ref[...], axis=-1, keepdims=True) + bmg_ref[...]

    cnorm = jnp.sqrt(jnp.sum(ck * ck, axis=-1, keepdims=True))
    ncand = ck / jnp.maximum(cnorm, _EPS)

    keys = keys_ref[...]
    R, N, _ = keys.shape
    keysq = jnp.sum(keys * keys, axis=-1)
    keynorm = jnp.maximum(jnp.sqrt(keysq), _EPS)
    dots = jnp.sum(keys * ncand[:, None, :], axis=-1)
    sim = dots / keynorm

    iota = lax.broadcasted_iota(jnp.int32, sim.shape, 1)
    neg = jnp.float32(-jnp.inf)

    s1 = jnp.max(sim, axis=-1, keepdims=True)
    i1 = jnp.argmax(sim, axis=-1, keepdims=True)
    sim_m = jnp.where(iota == i1, neg, sim)
    s2 = jnp.max(sim_m, axis=-1, keepdims=True)
    i2 = jnp.argmax(sim_m, axis=-1, keepdims=True)
    sim_m = jnp.where(iota == i2, neg, sim_m)
    s3 = jnp.max(sim_m, axis=-1, keepdims=True)
    i3 = jnp.argmax(sim_m, axis=-1, keepdims=True)

    strength = str_ref[...]
    age = age_ref[...]
    replace_scores = 1.2 * age + 1.0 * (1.0 - strength) + 0.5 * (1.0 - sim)
    ri = jnp.argmax(replace_scores, axis=-1, keepdims=True)

    novelty = jnp.clip(1.0 - s1, 0.0, 1.0)
    merge_pref = jax.nn.sigmoid(mg_lin + 2.6 * s1)
    full_m = (s1 > 0.78) & (merge_pref >= 0.55)
    multi_m = full_m & (s2 > 0.68)
    partial_m = (~multi_m) & (s1 > 0.64) & (s2 > 0.52)

    e2 = jnp.exp((s2 - s1) / _TEMPERATURE)
    e3 = jnp.exp((s3 - s1) / _TEMPERATURE)
    pden = 1.0 + e2
    pw1 = 1.0 / pden
    pw2 = e2 / pden
    mden = 1.0 + e2 + e3
    mw1 = 1.0 / mden
    mw2 = e2 / mden
    mw3 = e3 / mden

    oh1 = (iota == i1).astype(f32)
    oh2 = (iota == i2).astype(f32)
    oh3 = (iota == i3).astype(f32)
    tw = (iota == ri).astype(f32)
    tw = jnp.where(full_m, oh1, tw)
    tw = jnp.where(partial_m, pw1 * oh1 + pw2 * oh2, tw)
    tw = jnp.where(multi_m, mw1 * oh1 + mw2 * oh2 + mw3 * oh3, tw)

    scale = jnp.where(multi_m, 0.16 + 0.52 * ws,
                      jnp.where(partial_m, 0.18 + 0.62 * ws,
                                0.2 + 0.8 * ws))
    fac = scale * (0.55 + 0.45 * novelty)
    ow = tw * fac

    merge_like = full_m | partial_m | multi_m
    kmix = jnp.where(merge_like, 0.28 + 0.24 * pg, 0.78 + 0.16 * pg)
    vmix = jnp.where(merge_like, 0.42 + 0.28 * pg, 0.82 + 0.12 * pg)

    owk = (ow * kmix)[:, :, None]
    okeys_ref[...] = keys + owk * (ck[:, None, :] - keys)
    ovals_ref[...] = vals_ref[...]

    boost = ow * (0.45 + 0.35 * pg + 0.45 * novelty + 0.25 * ws)
    ostr_ref[...] = jnp.clip(strength * _STRENGTH_DECAY + boost, 0.0, 1.0)
    oage_ref[...] = jnp.clip((age + _AGE_INCREMENT) * (1.0 - ow), 0.0, 1.0)

    zero = jnp.zeros_like(s1)
    t1 = jnp.where(multi_m, mw1,
                   jnp.where(partial_m, pw1,
                             jnp.where(full_m, jnp.ones_like(s1), zero)))
    t2 = jnp.where(multi_m, mw2, jnp.where(partial_m, pw2, zero))
    t3 = jnp.where(multi_m, mw3, zero)
    tr = jnp.where(merge_like, zero, jnp.ones_like(s1))
    fv = fac * vmix
    c1, c2, c3, cr = t1 * fv, t2 * fv, t3 * fv, tr * fv

    def _used(c):
        return (i1 == c) | (i2 == c) | (i3 == c) | (ri == c)
    s_sent = jnp.where(~_used(0), 0,
                       jnp.where(~_used(1), 1,
                                 jnp.where(~_used(2), 2,
                                           jnp.where(~_used(3), 3, 4))))
    s_sent = s_sent.astype(jnp.int32)

    row_id = (pl.program_id(0) * R
              + lax.broadcasted_iota(jnp.int32, (R, 1), 0))
    base = row_id * N
    j1 = base + jnp.where(c1 > 0, i1, s_sent)
    j2 = base + jnp.where(c2 > 0, i2, s_sent)
    j3 = base + jnp.where(c3 > 0, i3, s_sent)
    j4 = base + jnp.where(cr > 0, ri, s_sent)
    jp = base + s_sent
    fidx_ref[...] = jnp.concatenate(
        [j1, j2, j3, j4, jp, jp, jp, jp], axis=1)

    z = jnp.zeros_like(c1)
    cs = jnp.concatenate([c1, c2, c3, cr, z, z, z, z], axis=1)
    D_ = cv.shape[-1]
    am_ref[...] = jnp.broadcast_to((1.0 - cs)[:, :, None], (R, 8, D_))
    cva_ref[...] = cs[:, :, None] * cv[:, None, :]


def _make_sc_fixup(n_entries, D, interpret=False):
    per_w = n_entries // _NWORKERS
    mesh = plsc.VectorSubcoreMesh(core_axis_name="c", subcore_axis_name="s",
                                  num_cores=_NCORES,
                                  num_subcores=_NWORKERS // _NCORES)

    @functools.partial(
        pl.kernel,
        out_type=(),
        mesh=mesh,
        interpret=interpret,
        scratch_types=[
            pltpu.VMEM((per_w,), jnp.int32),
            pltpu.VMEM((per_w, D), jnp.float32),
            pltpu.VMEM((per_w, D), jnp.float32),
            pltpu.VMEM((per_w, D), jnp.float32),
            pltpu.SemaphoreType.DMA,
        ],
    )
    def sc_fixup(vflat_hbm, fidx_hbm, am_hbm, cva_hbm, out_ref,
                 idx_v, rows_v, am_v, cva_v, sem):
        wid = lax.axis_index("s") * _NCORES + lax.axis_index("c")
        base = wid * per_w
        pltpu.sync_copy(fidx_hbm.at[pl.ds(base, per_w)], idx_v)
        pltpu.async_copy(vflat_hbm.at[idx_v], rows_v, sem).wait()
        pltpu.sync_copy(am_hbm.at[pl.ds(base, per_w)], am_v)
        pltpu.sync_copy(cva_hbm.at[pl.ds(base, per_w)], cva_v)

        def body(e, carry):
            for j in range(D // 16):
                sl = (e, pl.ds(j * 16, 16))
                rows_v[sl] = am_v[sl] * rows_v[sl] + cva_v[sl]
            return carry

        lax.fori_loop(0, per_w, body, 0)
        pltpu.async_copy(rows_v, out_ref.at[idx_v], sem).wait()

    return sc_fixup


def kernel(signal, hidden, episodic_keys, episodic_values, episodic_strength,
           episodic_age, Wk, bk, Wv, bv, Wwg, bwg, Wpg, bpg, Wmg, bmg,
           interpret=False):
    B, N, D = episodic_keys.shape
    R = _BLK
    grid = (B // R,)

    row = lambda i: (i, 0)
    row3 = lambda i: (i, 0, 0)
    const2 = lambda i: (0, 0)

    in_specs = [
        pl.BlockSpec((R, D), row),
        pl.BlockSpec((R, D), row),
        pl.BlockSpec((R, N, D), row3),
        pl.BlockSpec((R, N, D), row3),
        pl.BlockSpec((R, N), row),
        pl.BlockSpec((R, N), row),
        pl.BlockSpec((D, 2 * D), const2),
        pl.BlockSpec((1, D), const2),
        pl.BlockSpec((D, 2 * D), const2),
        pl.BlockSpec((1, D), const2),
        pl.BlockSpec((1, 2 * D), const2),
        pl.BlockSpec((1, 1), const2),
        pl.BlockSpec((1, 2 * D), const2),
        pl.BlockSpec((1, 1), const2),
        pl.BlockSpec((1, 2 * D), const2),
        pl.BlockSpec((1, 1), const2),
    ]
    out_specs = [
        pl.BlockSpec((R, N, D), row3),
        pl.BlockSpec((R, N, D), row3),
        pl.BlockSpec((R, N), row),
        pl.BlockSpec((R, N), row),
        pl.BlockSpec((R, 8), row),
        pl.BlockSpec((R, 8, D), lambda i: (i, 0, 0)),
        pl.BlockSpec((R, 8, D), lambda i: (i, 0, 0)),
    ]
    out_shapes = [
        jax.ShapeDtypeStruct((B, N, D), jnp.float32),
        jax.ShapeDtypeStruct((B, N, D), jnp.float32),
        jax.ShapeDtypeStruct((B, N), jnp.float32),
        jax.ShapeDtypeStruct((B, N), jnp.float32),
        jax.ShapeDtypeStruct((B, 8), jnp.int32),
        jax.ShapeDtypeStruct((B, 8, D), jnp.float32),
        jax.ShapeDtypeStruct((B, 8, D), jnp.float32),
    ]

    vflat = episodic_values.reshape(B * N, D)

    okeys, vcopy, ostr, oage, fidx, am3, cva3 = pl.pallas_call(
        _tc_body,
        grid=grid,
        in_specs=in_specs,
        out_specs=out_specs,
        out_shape=out_shapes,
        compiler_params=pltpu.CompilerParams(
            dimension_semantics=("parallel",)),
        interpret=interpret,
    )(signal, hidden, episodic_keys, episodic_values,
      episodic_strength, episodic_age,
      Wk, bk.reshape(1, D), Wv, bv.reshape(1, D),
      Wwg, bwg.reshape(1, 1), Wpg, bpg.reshape(1, 1), Wmg, bmg.reshape(1, 1))

    fidx_flat = fidx.reshape(8 * B)
    am_flat = am3.reshape(8 * B, D)
    cva_flat = cva3.reshape(8 * B, D)

    sc_fixup = _make_sc_fixup(8 * B, D, interpret=interpret)
    vref = jax.new_ref(vcopy.reshape(B * N, D))
    sc_fixup(vflat, fidx_flat, am_flat, cva_flat, vref)
    updated_values = vref[...].reshape(B, N, D)

    return okeys, updated_values, ostr, oage


# speedup vs baseline: 20.8310x; 1.1358x over previous
import jax
import jax.numpy as jnp
from jax import lax
from jax.experimental import pallas as pl
from jax.experimental.pallas import tpu as pltpu

_STRENGTH_DECAY = 0.99
_AGE_INCREMENT = 0.02
_TEMPERATURE = 0.1
_EPS = 1e-6

_BLK = 16


def _body(sig_ref, hid_ref, keys_ref, vals_ref, str_ref, age_ref,
          wk_ref, bk_ref, wv_ref, bv_ref, wwg_ref, bwg_ref,
          wpg_ref, bpg_ref, wmg_ref, bmg_ref,
          okeys_ref, ovals_ref, ostr_ref, oage_ref):
    f32 = jnp.float32
    sig = sig_ref[...]
    hid = hid_ref[...]
    joined = jnp.concatenate([sig, hid], axis=-1)

    ck = jnp.tanh(
        lax.dot_general(joined, wk_ref[...], (((1,), (1,)), ((), ())),
                        preferred_element_type=f32) + bk_ref[...])
    cv = jnp.tanh(
        lax.dot_general(joined, wv_ref[...], (((1,), (1,)), ((), ())),
                        preferred_element_type=f32) + bv_ref[...])
    ws = jax.nn.sigmoid(
        jnp.sum(joined * wwg_ref[...], axis=-1, keepdims=True) + bwg_ref[...])
    pg = jax.nn.sigmoid(
        jnp.sum(joined * wpg_ref[...], axis=-1, keepdims=True) + bpg_ref[...])
    mg_lin = jnp.sum(joined * wmg_ref[...], axis=-1, keepdims=True) + bmg_ref[...]

    cnorm = jnp.sqrt(jnp.sum(ck * ck, axis=-1, keepdims=True))
    ncand = ck / jnp.maximum(cnorm, _EPS)

    keys = keys_ref[...]
    keysq = jnp.sum(keys * keys, axis=-1)
    keynorm = jnp.maximum(jnp.sqrt(keysq), _EPS)
    dots = jnp.sum(keys * ncand[:, None, :], axis=-1)
    sim = dots / keynorm

    n = sim.shape[-1]
    iota = lax.broadcasted_iota(jnp.int32, sim.shape, 1)
    neg = jnp.float32(-jnp.inf)

    s1 = jnp.max(sim, axis=-1, keepdims=True)
    i1 = jnp.argmax(sim, axis=-1, keepdims=True)
    sim_m = jnp.where(iota == i1, neg, sim)
    s2 = jnp.max(sim_m, axis=-1, keepdims=True)
    i2 = jnp.argmax(sim_m, axis=-1, keepdims=True)
    sim_m = jnp.where(iota == i2, neg, sim_m)
    s3 = jnp.max(sim_m, axis=-1, keepdims=True)
    i3 = jnp.argmax(sim_m, axis=-1, keepdims=True)

    strength = str_ref[...]
    age = age_ref[...]
    replace_scores = 1.2 * age + 1.0 * (1.0 - strength) + 0.5 * (1.0 - sim)
    ri = jnp.argmax(replace_scores, axis=-1, keepdims=True)

    novelty = jnp.clip(1.0 - s1, 0.0, 1.0)
    merge_pref = jax.nn.sigmoid(mg_lin + 2.6 * s1)
    full_m = (s1 > 0.78) & (merge_pref >= 0.55)
    multi_m = full_m & (s2 > 0.68)
    partial_m = (~multi_m) & (s1 > 0.64) & (s2 > 0.52)

    e2 = jnp.exp((s2 - s1) / _TEMPERATURE)
    e3 = jnp.exp((s3 - s1) / _TEMPERATURE)
    pden = 1.0 + e2
    pw1 = 1.0 / pden
    pw2 = e2 / pden
    mden = 1.0 + e2 + e3
    mw1 = 1.0 / mden
    mw2 = e2 / mden
    mw3 = e3 / mden

    oh1 = (iota == i1).astype(f32)
    oh2 = (iota == i2).astype(f32)
    oh3 = (iota == i3).astype(f32)
    tw = (iota == ri).astype(f32)
    tw = jnp.where(full_m, oh1, tw)
    tw = jnp.where(partial_m, pw1 * oh1 + pw2 * oh2, tw)
    tw = jnp.where(multi_m, mw1 * oh1 + mw2 * oh2 + mw3 * oh3, tw)

    scale = jnp.where(multi_m, 0.16 + 0.52 * ws,
                      jnp.where(partial_m, 0.18 + 0.62 * ws,
                                0.2 + 0.8 * ws))
    ow = tw * (scale * (0.55 + 0.45 * novelty))

    merge_like = full_m | partial_m | multi_m
    kmix = jnp.where(merge_like, 0.28 + 0.24 * pg, 0.78 + 0.16 * pg)
    vmix = jnp.where(merge_like, 0.42 + 0.28 * pg, 0.82 + 0.12 * pg)

    owk = (ow * kmix)[:, :, None]
    okeys_ref[...] = keys + owk * (ck[:, None, :] - keys)
    vals = vals_ref[...]
    owv = (ow * vmix)[:, :, None]
    ovals_ref[...] = vals + owv * (cv[:, None, :] - vals)

    boost = ow * (0.45 + 0.35 * pg + 0.45 * novelty + 0.25 * ws)
    ostr_ref[...] = jnp.clip(strength * _STRENGTH_DECAY + boost, 0.0, 1.0)
    oage_ref[...] = jnp.clip((age + _AGE_INCREMENT) * (1.0 - ow), 0.0, 1.0)


def kernel(signal, hidden, episodic_keys, episodic_values, episodic_strength,
           episodic_age, Wk, bk, Wv, bv, Wwg, bwg, Wpg, bpg, Wmg, bmg,
           interpret=False):
    B, N, D = episodic_keys.shape
    R = _BLK
    grid = (B // R,)

    row = lambda i: (i, 0)
    row3 = lambda i: (i, 0, 0)
    const2 = lambda i: (0, 0)

    in_specs = [
        pl.BlockSpec((R, D), row),
        pl.BlockSpec((R, D), row),
        pl.BlockSpec((R, N, D), row3),
        pl.BlockSpec((R, N, D), row3),
        pl.BlockSpec((R, N), row),
        pl.BlockSpec((R, N), row),
        pl.BlockSpec((D, 2 * D), const2),
        pl.BlockSpec((1, D), const2),
        pl.BlockSpec((D, 2 * D), const2),
        pl.BlockSpec((1, D), const2),
        pl.BlockSpec((1, 2 * D), const2),
        pl.BlockSpec((1, 1), const2),
        pl.BlockSpec((1, 2 * D), const2),
        pl.BlockSpec((1, 1), const2),
        pl.BlockSpec((1, 2 * D), const2),
        pl.BlockSpec((1, 1), const2),
    ]
    out_specs = [
        pl.BlockSpec((R, N, D), row3),
        pl.BlockSpec((R, N, D), row3),
        pl.BlockSpec((R, N), row),
        pl.BlockSpec((R, N), row),
    ]
    out_shapes = [
        jax.ShapeDtypeStruct((B, N, D), jnp.float32),
        jax.ShapeDtypeStruct((B, N, D), jnp.float32),
        jax.ShapeDtypeStruct((B, N), jnp.float32),
        jax.ShapeDtypeStruct((B, N), jnp.float32),
    ]

    out = pl.pallas_call(
        _body,
        grid=grid,
        in_specs=in_specs,
        out_specs=out_specs,
        out_shape=out_shapes,
        compiler_params=pltpu.CompilerParams(
            dimension_semantics=("parallel",)),
        interpret=interpret,
    )(signal, hidden, episodic_keys, episodic_values,
      episodic_strength, episodic_age,
      Wk, bk.reshape(1, D), Wv, bv.reshape(1, D),
      Wwg, bwg.reshape(1, 1), Wpg, bpg.reshape(1, 1), Wmg, bmg.reshape(1, 1))
    return tuple(out)
